# baseline (speedup 1.0000x reference)
import functools

import jax
import jax.numpy as jnp
from jax import lax
from jax.experimental import pallas as pl
from jax.experimental.pallas import tpu as pltpu
from jax.experimental.pallas import tpu_sc as plsc

NU = 25000
NN = 50000
NE = 800000
D = 64
HD = 32
NLAYER = 3
BATCH = 16384

NC = 2
NS = 16
CH = 128
CPB = 12
NB = 34
CPS = CPB * NB
EPS = CPS * CH
EP = EPS * NS
RPS = 3128
RLAST = NN - (NS - 1) * RPS
NNP = NS * RPS

@functools.cache
def _sc_mesh():
    return plsc.VectorSubcoreMesh(core_axis_name="c", subcore_axis_name="s")


NBUF = 6
LA = 5
NSB = 3
HW = HD // 2
WBC = 8


def _layer_body(emb_hbm, src_hbm, dst_hbm, val_hbm, zer_hbm,
                out_hbm,
                src_v, dst_v, val_v, *bufs_and_sems):
    bufs = list(bufs_and_sems[:NBUF])
    stag = list(bufs_and_sems[NBUF:NBUF + NSB])
    acc = bufs_and_sems[NBUF + NSB]
    gsems = list(bufs_and_sems[NBUF + NSB + 1:NBUF + NSB + 1 + NBUF])
    ssems = list(bufs_and_sems[NBUF + NSB + 1 + NBUF:])
    c = lax.axis_index("c")
    s = lax.axis_index("s")

    pltpu.sync_copy(zer_hbm, acc.at[pl.ds(s * RPS, RPS)])
    plsc.subcore_barrier()

    def scale(rows, out, j):
        def grp_body(g, cc):
            vv = val_v[j, pl.ds(g * 16, 16)]
            for k in range(16):
                v = vv[k]
                r = g * 16 + k
                w = plsc.bitcast(rows[r, 0:HW], jnp.bfloat16)
                a, b = plsc.unpack(w, format=plsc.PackFormat.INTERLEAVED)
                out[r, 0:16] = a * v
                out[r, 16:32] = b * v
            return cc

        lax.fori_loop(0, CH // 16, grp_body, 0, unroll=2)

    def block_body(b, carry):
        i = s * NB + b
        pltpu.sync_copy(src_hbm.at[c * (NS * NB) + i], src_v)
        pltpu.sync_copy(dst_hbm.at[i], dst_v)
        pltpu.sync_copy(val_hbm.at[i], val_v)
        g = {}
        sc = {}
        for t in range(LA):
            g[t] = pltpu.async_copy(emb_hbm.at[src_v.at[t]],
                                    bufs[t], gsems[t])
        for j in range(CPB):
            p = j % NBUF
            m = j % NSB
            t = j + LA
            if t < CPB:
                q = t % NBUF
                g[t] = pltpu.async_copy(emb_hbm.at[src_v.at[t]],
                                        bufs[q], gsems[q])
            g[j].wait()
            if j >= NSB:
                sc[j - NSB].wait()
            scale(bufs[p], stag[m], j)
            sc[j] = pltpu.async_copy(stag[m], acc.at[dst_v.at[j]],
                                     ssems[m], add=True)
        for j in range(CPB - NSB, CPB):
            sc[j].wait()
        return carry

    lax.fori_loop(0, NB, block_body, 0)
    plsc.subcore_barrier()

    @pl.when(s < NS - 1)
    def _():
        pltpu.sync_copy(acc.at[pl.ds(s * RPS, RPS)],
                        out_hbm.at[pl.ds(c * NN + s * RPS, RPS)])

    @pl.when(s == NS - 1)
    def _():
        pltpu.sync_copy(acc.at[pl.ds((NS - 1) * RPS, RLAST)],
                        out_hbm.at[pl.ds(c * NN + (NS - 1) * RPS, RLAST)])


@functools.cache
def _layer_kernel():
    return pl.kernel(
        _layer_body,
        out_type=jax.ShapeDtypeStruct((2 * NN, HD), jnp.float32),
        mesh=_sc_mesh(),
        scratch_types=(
            [pltpu.VMEM((CPB, CH), jnp.int32),
             pltpu.VMEM((CPB, CH), jnp.int32),
             pltpu.VMEM((CPB, CH), jnp.float32)]
            + [pltpu.VMEM((CH, HW), jnp.int32) for _ in range(NBUF)]
            + [pltpu.VMEM((CH, HD), jnp.float32) for _ in range(NSB)]
            + [pltpu.VMEM_SHARED((NNP, HD), jnp.float32)]
            + [pltpu.SemaphoreType.DMA for _ in range(NBUF + NSB)]
        ),
        compiler_params=pltpu.CompilerParams(use_tc_tiling_on_sc=False,
                                             needs_layout_passes=False),
    )


PBN = 2000


def _pack_body(x, o):
    v = x[...]
    au = lax.bitcast_convert_type(v[:, 0:16].astype(jnp.bfloat16),
                                  jnp.uint16).astype(jnp.uint32)
    bu = lax.bitcast_convert_type(v[:, 16:32].astype(jnp.bfloat16),
                                  jnp.uint16).astype(jnp.uint32)
    o[...] = ((bu << 16) | au).astype(jnp.int32)


def _pack_call(x):
    return pl.pallas_call(
        _pack_body,
        grid=(2 * NN // PBN,),
        in_specs=[pl.BlockSpec((PBN, HD), lambda i: (i, 0))],
        out_specs=pl.BlockSpec((PBN, HW), lambda i: (i, 0)),
        out_shape=jax.ShapeDtypeStruct((2 * NN, HW), jnp.int32),
    )(x)


BN = 2000
GN = NN // BN


def _gate_body(p_ref, l0, h0, l1, h1, l2, h2, l3, h3, olo, ohi):
    a0 = l0[...]
    b0 = h0[...]
    sq = (jnp.sum(a0 * a0, axis=1, keepdims=True)
          + jnp.sum(b0 * b0, axis=1, keepdims=True))
    nrm = jnp.sqrt(sq)
    mn = (jnp.sum(a0, axis=1, keepdims=True)
          + jnp.sum(b0, axis=1, keepdims=True)) * (1.0 / D)
    lg = [nrm * p_ref[l] + mn * p_ref[4 + l] + p_ref[8 + l] for l in range(4)]
    m = jnp.maximum(jnp.maximum(lg[0], lg[1]), jnp.maximum(lg[2], lg[3]))
    ex = [jnp.exp(x - m) for x in lg]
    den = ex[0] + ex[1] + ex[2] + ex[3]
    al = [e / den for e in ex]
    los = [a0, l1[...], l2[...], l3[...]]
    his = [b0, h1[...], h2[...], h3[...]]
    olo[...] = al[0] * los[0] + al[1] * los[1] + al[2] * los[2] + al[3] * los[3]
    ohi[...] = al[0] * his[0] + al[1] * his[1] + al[2] * his[2] + al[3] * his[3]


def _gate_call(params, e0, e1, e2, e3):
    lo_spec = pl.BlockSpec((BN, HD), lambda i: (i, 0))
    hi_spec = pl.BlockSpec((BN, HD), lambda i: (i + GN, 0))
    return pl.pallas_call(
        _gate_body,
        grid=(GN,),
        in_specs=[pl.BlockSpec(memory_space=pltpu.SMEM),
                  lo_spec, hi_spec, lo_spec, hi_spec,
                  lo_spec, hi_spec, lo_spec, hi_spec],
        out_specs=[pl.BlockSpec((BN, HD), lambda i: (i, 0)),
                   pl.BlockSpec((BN, HD), lambda i: (i, 0))],
        out_shape=[jax.ShapeDtypeStruct((NN, HD), jnp.float32),
                   jax.ShapeDtypeStruct((NN, HD), jnp.float32)],
    )(params, e0, e0, e1, e1, e2, e2, e3, e3)


CPW = BATCH // CH // (NC * NS)
EPW = CPW * CH


def _batch_gather_body(lo, hi, uix, iix, ulo, uhi, ilo, ihi, idx_v, buf):
    c = lax.axis_index("c")
    s = lax.axis_index("s")
    w = s * NC + c
    pltpu.sync_copy(uix.at[w], idx_v)
    for j in range(CPW):
        pltpu.sync_copy(lo.at[idx_v.at[j]], buf)
        pltpu.sync_copy(buf, ulo.at[pl.ds(w * EPW + j * CH, CH)])
        pltpu.sync_copy(hi.at[idx_v.at[j]], buf)
        pltpu.sync_copy(buf, uhi.at[pl.ds(w * EPW + j * CH, CH)])
    pltpu.sync_copy(iix.at[w], idx_v)
    for j in range(CPW):
        pltpu.sync_copy(lo.at[idx_v.at[j]], buf)
        pltpu.sync_copy(buf, ilo.at[pl.ds(w * EPW + j * CH, CH)])
        pltpu.sync_copy(hi.at[idx_v.at[j]], buf)
        pltpu.sync_copy(buf, ihi.at[pl.ds(w * EPW + j * CH, CH)])


@functools.cache
def _batch_gather_kernel():
    return pl.kernel(
        _batch_gather_body,
        out_type=tuple(jax.ShapeDtypeStruct((BATCH, HD), jnp.float32)
                       for _ in range(4)),
        mesh=_sc_mesh(),
        scratch_types=[
            pltpu.VMEM((CPW, CH), jnp.int32),
            pltpu.VMEM((CH, HD), jnp.float32),
        ],
        compiler_params=pltpu.CompilerParams(use_tc_tiling_on_sc=False),
    )


def _dot_body(ul, il, uh, ih, o):
    d = jnp.sum(ul[...] * il[...] + uh[...] * ih[...], axis=1)
    o[...] = d.reshape(BATCH // 2048, 2048)


def _dot_call(ulo, ilo, uhi, ihi):
    out = pl.pallas_call(
        _dot_body,
        out_shape=jax.ShapeDtypeStruct((BATCH // 2048, 2048), jnp.float32),
    )(ulo, ilo, uhi, ihi)
    return out.reshape(BATCH)


def kernel(users, items, user_emb, item_emb, gate_w, gate_b,
           edge_src, edge_dst, edge_val):
    pad = EP - NE
    src_p = jnp.concatenate([edge_src, jnp.zeros((pad,), jnp.int32)])
    dst_p = jnp.concatenate([edge_dst, jnp.zeros((pad,), jnp.int32)])
    val_p = jnp.concatenate([edge_val, jnp.zeros((pad,), jnp.float32)])
    src2 = jnp.concatenate([src_p, src_p + NN]).reshape(2 * NS * NB, CPB, CH)
    dst2 = dst_p.reshape(NS * NB, CPB, CH)
    val2 = val_p.reshape(NS * NB, CPB, CH)
    zer = jnp.zeros((RPS, HD), jnp.float32)

    emb0 = jnp.concatenate([user_emb, item_emb], axis=0)
    e0 = jnp.concatenate([emb0[:, :HD], emb0[:, HD:]], axis=0)

    layer = _layer_kernel()
    e1 = layer(_pack_call(e0), src2, dst2, val2, zer)
    e2 = layer(_pack_call(e1), src2, dst2, val2, zer)
    e3 = layer(_pack_call(e2), src2, dst2, val2, zer)

    params = jnp.concatenate([gate_w[:, 0], gate_w[:, 1], gate_b])
    olo, ohi = _gate_call(params, e0, e1, e2, e3)

    uix = users.reshape(NC * NS, CPW, CH)
    iix = (items + NU).reshape(NC * NS, CPW, CH)
    ulo, uhi, ilo, ihi = _batch_gather_kernel()(olo, ohi, uix, iix)

    return _dot_call(ulo, ilo, uhi, ihi)


# speedup vs baseline: 1.1106x; 1.1106x over previous
import functools

import jax
import jax.numpy as jnp
from jax import lax
from jax.experimental import pallas as pl
from jax.experimental.pallas import tpu as pltpu
from jax.experimental.pallas import tpu_sc as plsc

NU = 25000
NN = 50000
NE = 800000
D = 64
HD = 32
NLAYER = 3
BATCH = 16384

NC = 2
NS = 16
CH = 128
CPB = 24
NB = 17
CPS = CPB * NB
EPS = CPS * CH
EP = EPS * NS
RPS = 3128
RLAST = NN - (NS - 1) * RPS
NNP = NS * RPS

@functools.cache
def _sc_mesh():
    return plsc.VectorSubcoreMesh(core_axis_name="c", subcore_axis_name="s")


NBUF = 5
LA = 4
NSB = 2
HW = HD // 2
WBC = 8


def _layer_body(emb_hbm, src_hbm, dst_hbm, val_hbm, zer_hbm,
                out_hbm,
                src_v, dst_v, val_v, *bufs_and_sems):
    bufs = list(bufs_and_sems[:NBUF])
    stag = list(bufs_and_sems[NBUF:NBUF + NSB])
    acc = bufs_and_sems[NBUF + NSB]
    gsems = list(bufs_and_sems[NBUF + NSB + 1:NBUF + NSB + 1 + NBUF])
    ssems = list(bufs_and_sems[NBUF + NSB + 1 + NBUF:])
    c = lax.axis_index("c")
    s = lax.axis_index("s")

    pltpu.sync_copy(zer_hbm, acc.at[pl.ds(s * RPS, RPS)])
    plsc.subcore_barrier()

    def scale(rows, out, j):
        def grp_body(g, cc):
            vv = val_v[j, pl.ds(g * 16, 16)]
            for k in range(16):
                v = vv[k]
                r = g * 16 + k
                w = plsc.bitcast(rows[r, 0:HW], jnp.bfloat16)
                a, b = plsc.unpack(w, format=plsc.PackFormat.INTERLEAVED)
                out[r, 0:16] = a * v
                out[r, 16:32] = b * v
            return cc

        lax.fori_loop(0, CH // 16, grp_body, 0, unroll=2)

    def block_body(b, carry):
        i = s * NB + b
        pltpu.sync_copy(src_hbm.at[c * (NS * NB) + i], src_v)
        pltpu.sync_copy(dst_hbm.at[i], dst_v)
        pltpu.sync_copy(val_hbm.at[i], val_v)
        g = {}
        sc = {}
        for t in range(LA):
            g[t] = pltpu.async_copy(emb_hbm.at[src_v.at[t]],
                                    bufs[t], gsems[t])
        for j in range(CPB):
            p = j % NBUF
            m = j % NSB
            t = j + LA
            if t < CPB:
                q = t % NBUF
                g[t] = pltpu.async_copy(emb_hbm.at[src_v.at[t]],
                                        bufs[q], gsems[q])
            g[j].wait()
            if j >= NSB:
                sc[j - NSB].wait()
            scale(bufs[p], stag[m], j)
            sc[j] = pltpu.async_copy(stag[m], acc.at[dst_v.at[j]],
                                     ssems[m], add=True)
        for j in range(CPB - NSB, CPB):
            sc[j].wait()
        return carry

    lax.fori_loop(0, NB, block_body, 0)
    plsc.subcore_barrier()

    @pl.when(s < NS - 1)
    def _():
        pltpu.sync_copy(acc.at[pl.ds(s * RPS, RPS)],
                        out_hbm.at[pl.ds(c * NN + s * RPS, RPS)])

    @pl.when(s == NS - 1)
    def _():
        pltpu.sync_copy(acc.at[pl.ds((NS - 1) * RPS, RLAST)],
                        out_hbm.at[pl.ds(c * NN + (NS - 1) * RPS, RLAST)])


@functools.cache
def _layer_kernel():
    return pl.kernel(
        _layer_body,
        out_type=jax.ShapeDtypeStruct((2 * NN, HD), jnp.float32),
        mesh=_sc_mesh(),
        scratch_types=(
            [pltpu.VMEM((CPB, CH), jnp.int32),
             pltpu.VMEM((CPB, CH), jnp.int32),
             pltpu.VMEM((CPB, CH), jnp.float32)]
            + [pltpu.VMEM((CH, HW), jnp.int32) for _ in range(NBUF)]
            + [pltpu.VMEM((CH, HD), jnp.float32) for _ in range(NSB)]
            + [pltpu.VMEM_SHARED((NNP, HD), jnp.float32)]
            + [pltpu.SemaphoreType.DMA for _ in range(NBUF + NSB)]
        ),
        compiler_params=pltpu.CompilerParams(use_tc_tiling_on_sc=False,
                                             needs_layout_passes=False),
    )


PBN = 2000


def _pack_body(x, o):
    v = x[...]
    au = lax.bitcast_convert_type(v[:, 0:16].astype(jnp.bfloat16),
                                  jnp.uint16).astype(jnp.uint32)
    bu = lax.bitcast_convert_type(v[:, 16:32].astype(jnp.bfloat16),
                                  jnp.uint16).astype(jnp.uint32)
    o[...] = ((bu << 16) | au).astype(jnp.int32)


def _pack_call(x):
    return pl.pallas_call(
        _pack_body,
        grid=(2 * NN // PBN,),
        in_specs=[pl.BlockSpec((PBN, HD), lambda i: (i, 0))],
        out_specs=pl.BlockSpec((PBN, HW), lambda i: (i, 0)),
        out_shape=jax.ShapeDtypeStruct((2 * NN, HW), jnp.int32),
    )(x)


BN = 2000
GN = NN // BN


def _gate_body(p_ref, l0, h0, l1, h1, l2, h2, l3, h3, olo, ohi):
    a0 = l0[...]
    b0 = h0[...]
    sq = (jnp.sum(a0 * a0, axis=1, keepdims=True)
          + jnp.sum(b0 * b0, axis=1, keepdims=True))
    nrm = jnp.sqrt(sq)
    mn = (jnp.sum(a0, axis=1, keepdims=True)
          + jnp.sum(b0, axis=1, keepdims=True)) * (1.0 / D)
    lg = [nrm * p_ref[l] + mn * p_ref[4 + l] + p_ref[8 + l] for l in range(4)]
    m = jnp.maximum(jnp.maximum(lg[0], lg[1]), jnp.maximum(lg[2], lg[3]))
    ex = [jnp.exp(x - m) for x in lg]
    den = ex[0] + ex[1] + ex[2] + ex[3]
    al = [e / den for e in ex]
    los = [a0, l1[...], l2[...], l3[...]]
    his = [b0, h1[...], h2[...], h3[...]]
    olo[...] = al[0] * los[0] + al[1] * los[1] + al[2] * los[2] + al[3] * los[3]
    ohi[...] = al[0] * his[0] + al[1] * his[1] + al[2] * his[2] + al[3] * his[3]


def _gate_call(params, e0, e1, e2, e3):
    lo_spec = pl.BlockSpec((BN, HD), lambda i: (i, 0))
    hi_spec = pl.BlockSpec((BN, HD), lambda i: (i + GN, 0))
    return pl.pallas_call(
        _gate_body,
        grid=(GN,),
        in_specs=[pl.BlockSpec(memory_space=pltpu.SMEM),
                  lo_spec, hi_spec, lo_spec, hi_spec,
                  lo_spec, hi_spec, lo_spec, hi_spec],
        out_specs=[pl.BlockSpec((BN, HD), lambda i: (i, 0)),
                   pl.BlockSpec((BN, HD), lambda i: (i, 0))],
        out_shape=[jax.ShapeDtypeStruct((NN, HD), jnp.float32),
                   jax.ShapeDtypeStruct((NN, HD), jnp.float32)],
    )(params, e0, e0, e1, e1, e2, e2, e3, e3)


CPW = BATCH // CH // (NC * NS)
EPW = CPW * CH


def _batch_gather_body(lo, hi, uix, iix, ulo, uhi, ilo, ihi, idx_v, buf):
    c = lax.axis_index("c")
    s = lax.axis_index("s")
    w = s * NC + c
    pltpu.sync_copy(uix.at[w], idx_v)
    for j in range(CPW):
        pltpu.sync_copy(lo.at[idx_v.at[j]], buf)
        pltpu.sync_copy(buf, ulo.at[pl.ds(w * EPW + j * CH, CH)])
        pltpu.sync_copy(hi.at[idx_v.at[j]], buf)
        pltpu.sync_copy(buf, uhi.at[pl.ds(w * EPW + j * CH, CH)])
    pltpu.sync_copy(iix.at[w], idx_v)
    for j in range(CPW):
        pltpu.sync_copy(lo.at[idx_v.at[j]], buf)
        pltpu.sync_copy(buf, ilo.at[pl.ds(w * EPW + j * CH, CH)])
        pltpu.sync_copy(hi.at[idx_v.at[j]], buf)
        pltpu.sync_copy(buf, ihi.at[pl.ds(w * EPW + j * CH, CH)])


@functools.cache
def _batch_gather_kernel():
    return pl.kernel(
        _batch_gather_body,
        out_type=tuple(jax.ShapeDtypeStruct((BATCH, HD), jnp.float32)
                       for _ in range(4)),
        mesh=_sc_mesh(),
        scratch_types=[
            pltpu.VMEM((CPW, CH), jnp.int32),
            pltpu.VMEM((CH, HD), jnp.float32),
        ],
        compiler_params=pltpu.CompilerParams(use_tc_tiling_on_sc=False),
    )


def _dot_body(ul, il, uh, ih, o):
    d = jnp.sum(ul[...] * il[...] + uh[...] * ih[...], axis=1)
    o[...] = d.reshape(BATCH // 2048, 2048)


def _dot_call(ulo, ilo, uhi, ihi):
    out = pl.pallas_call(
        _dot_body,
        out_shape=jax.ShapeDtypeStruct((BATCH // 2048, 2048), jnp.float32),
    )(ulo, ilo, uhi, ihi)
    return out.reshape(BATCH)


def kernel(users, items, user_emb, item_emb, gate_w, gate_b,
           edge_src, edge_dst, edge_val):
    pad = EP - NE
    src_p = jnp.concatenate([edge_src, jnp.zeros((pad,), jnp.int32)])
    dst_p = jnp.concatenate([edge_dst, jnp.zeros((pad,), jnp.int32)])
    val_p = jnp.concatenate([edge_val, jnp.zeros((pad,), jnp.float32)])
    src2 = jnp.concatenate([src_p, src_p + NN]).reshape(2 * NS * NB, CPB, CH)
    dst2 = dst_p.reshape(NS * NB, CPB, CH)
    val2 = val_p.reshape(NS * NB, CPB, CH)
    zer = jnp.zeros((RPS, HD), jnp.float32)

    emb0 = jnp.concatenate([user_emb, item_emb], axis=0)
    e0 = jnp.concatenate([emb0[:, :HD], emb0[:, HD:]], axis=0)

    layer = _layer_kernel()
    e1 = layer(_pack_call(e0), src2, dst2, val2, zer)
    e2 = layer(_pack_call(e1), src2, dst2, val2, zer)
    e3 = layer(_pack_call(e2), src2, dst2, val2, zer)

    params = jnp.concatenate([gate_w[:, 0], gate_w[:, 1], gate_b])
    olo, ohi = _gate_call(params, e0, e1, e2, e3)

    uix = users.reshape(NC * NS, CPW, CH)
    iix = (items + NU).reshape(NC * NS, CPW, CH)
    ulo, uhi, ilo, ihi = _batch_gather_kernel()(olo, ohi, uix, iix)

    return _dot_call(ulo, ilo, uhi, ihi)
